# baseline (speedup 1.0000x reference)
import functools

import jax
import jax.numpy as jnp
from jax import lax
from jax.experimental import pallas as pl
from jax.experimental.pallas import tpu as pltpu
from jax.experimental.pallas import tpu_sc as plsc

N_DISTS = 100000
N_BITS = 16
HIDDEN = 2 * N_BITS
D = HIDDEN * N_BITS
B = 16384
NC, NS = 2, 16
NW = NC * NS
FPW = D // NW
SEG = 2048
XMAIN = B - SEG
L = 16


def _make_gather():
    mesh = plsc.VectorSubcoreMesh(core_axis_name="c", subcore_axis_name="s")

    @functools.partial(
        pl.kernel,
        mesh=mesh,
        out_type=[
            jax.ShapeDtypeStruct((D, B), jnp.float32),
            jax.ShapeDtypeStruct((HIDDEN, B), jnp.float32),
        ],
        scratch_types=[
            pltpu.VMEM((XMAIN,), jnp.int32),
            pltpu.VMEM((SEG,), jnp.float32),
            pltpu.VMEM((N_DISTS,), jnp.float32),
            pltpu.VMEM((XMAIN,), jnp.float32),
            pltpu.SemaphoreType.DMA,
            pltpu.SemaphoreType.DMA,
            pltpu.SemaphoreType.DMA,
        ],
        compiler_params=pltpu.CompilerParams(needs_layout_passes=False),
    )
    def gather_kernel(x_hbm, xf_hbm, wst_hbm, rst_hbm, w_out, r_out,
                      x_v, x7, rowbuf, obuf, sm, st, sx):
        wid = lax.axis_index("s") * NC + lax.axis_index("c")
        row0 = wid * FPW

        pltpu.sync_copy(x_hbm.at[pl.ds(0, XMAIN)], x_v)
        pltpu.async_copy(obuf, w_out.at[row0, pl.ds(0, XMAIN)], sm)
        pltpu.async_copy(obuf.at[pl.ds(0, SEG)],
                         w_out.at[row0, pl.ds(XMAIN, SEG)], st)

        def gather_main():
            @plsc.parallel_loop(0, XMAIN, step=L, unroll=12)
            def body(i):
                idx = x_v[pl.ds(i, L)]
                obuf[pl.ds(i, L)] = plsc.load_gather(rowbuf, [idx])

        def gather_tail():
            @plsc.parallel_loop(0, SEG, step=L, unroll=12)
            def body(i):
                idx = plsc.bitcast(x7[pl.ds(i, L)], jnp.int32)
                x7[pl.ds(i, L)] = plsc.load_gather(rowbuf, [idx])

        def drain_writes():
            pltpu.make_async_copy(
                w_out.at[row0, pl.ds(0, XMAIN)], obuf, sm).wait()
            pltpu.make_async_copy(
                w_out.at[row0, pl.ds(XMAIN, SEG)], obuf.at[pl.ds(0, SEG)],
                st).wait()

        def do_row(row, src_t, out_t):
            pltpu.sync_copy(src_t.at[row], rowbuf)
            drain_writes()
            restage = pltpu.async_copy(xf_hbm.at[pl.ds(XMAIN, SEG)], x7, sx)
            gather_main()
            restage.wait()
            gather_tail()
            pltpu.async_copy(obuf, out_t.at[row, pl.ds(0, XMAIN)], sm)
            pltpu.async_copy(x7, out_t.at[row, pl.ds(XMAIN, SEG)], st)

        def wrows(k, _):
            do_row(row0 + k, wst_hbm, w_out)
            return 0

        lax.fori_loop(0, FPW, wrows, 0)
        do_row(wid, rst_hbm, r_out)
        drain_writes()

    return gather_kernel


_gather = _make_gather()


def kernel(x, Ws, rs):
    x32 = x.astype(jnp.int32)
    xf = lax.bitcast_convert_type(x32, jnp.float32)
    Wt = Ws.transpose(1, 2, 0).reshape(D, N_DISTS)
    rt = rs.transpose(1, 0)
    OW, OR = _gather(x32, xf, Wt, rt)
    W = OW.reshape(HIDDEN, N_BITS, B).transpose(2, 0, 1)
    r = OR.transpose(1, 0)
    return (W, r)


# speedup vs baseline: 1.0117x; 1.0117x over previous
import functools

import jax
import jax.numpy as jnp
from jax import lax
from jax.experimental import pallas as pl
from jax.experimental.pallas import tpu as pltpu
from jax.experimental.pallas import tpu_sc as plsc

N_DISTS = 100000
N_BITS = 16
HIDDEN = 2 * N_BITS
D = HIDDEN * N_BITS
B = 16384
NC, NS = 2, 16
NW = NC * NS
FPW = D // NW
SEG = 2048
XMAIN = B - SEG
L = 16


def _make_gather():
    mesh = plsc.VectorSubcoreMesh(core_axis_name="c", subcore_axis_name="s")

    @functools.partial(
        pl.kernel,
        mesh=mesh,
        out_type=[
            jax.ShapeDtypeStruct((D, B), jnp.float32),
            jax.ShapeDtypeStruct((HIDDEN, B), jnp.float32),
        ],
        scratch_types=[
            pltpu.VMEM((XMAIN,), jnp.int32),
            pltpu.VMEM((SEG,), jnp.float32),
            pltpu.VMEM((N_DISTS,), jnp.float32),
            pltpu.VMEM((XMAIN,), jnp.float32),
            pltpu.SemaphoreType.DMA,
            pltpu.SemaphoreType.DMA,
            pltpu.SemaphoreType.DMA,
        ],
        compiler_params=pltpu.CompilerParams(needs_layout_passes=False),
    )
    def gather_kernel(x_hbm, xf_hbm, wst_hbm, rst_hbm, w_out, r_out,
                      x_v, x7, rowbuf, obuf, sm, st, sx):
        wid = lax.axis_index("s") * NC + lax.axis_index("c")
        row0 = wid * FPW

        pltpu.sync_copy(x_hbm.at[pl.ds(0, XMAIN)], x_v)
        pltpu.async_copy(obuf, w_out.at[row0, pl.ds(0, XMAIN)], sm)
        pltpu.async_copy(obuf.at[pl.ds(0, SEG)],
                         w_out.at[row0, pl.ds(XMAIN, SEG)], st)

        def gather_main():
            @plsc.parallel_loop(0, XMAIN, step=L, unroll=8)
            def body(i):
                idx = x_v[pl.ds(i, L)]
                obuf[pl.ds(i, L)] = plsc.load_gather(rowbuf, [idx])

        def gather_tail():
            @plsc.parallel_loop(0, SEG, step=L, unroll=8)
            def body(i):
                idx = plsc.bitcast(x7[pl.ds(i, L)], jnp.int32)
                x7[pl.ds(i, L)] = plsc.load_gather(rowbuf, [idx])

        def drain_writes():
            pltpu.make_async_copy(
                w_out.at[row0, pl.ds(0, XMAIN)], obuf, sm).wait()
            pltpu.make_async_copy(
                w_out.at[row0, pl.ds(XMAIN, SEG)], obuf.at[pl.ds(0, SEG)],
                st).wait()

        def do_row(row, src_t, out_t):
            pltpu.sync_copy(src_t.at[row], rowbuf)
            drain_writes()
            restage = pltpu.async_copy(xf_hbm.at[pl.ds(XMAIN, SEG)], x7, sx)
            gather_main()
            restage.wait()
            gather_tail()
            pltpu.async_copy(obuf, out_t.at[row, pl.ds(0, XMAIN)], sm)
            pltpu.async_copy(x7, out_t.at[row, pl.ds(XMAIN, SEG)], st)

        def wrows(k, _):
            do_row(row0 + k, wst_hbm, w_out)
            return 0

        lax.fori_loop(0, FPW, wrows, 0)
        do_row(wid, rst_hbm, r_out)
        drain_writes()

    return gather_kernel


_gather = _make_gather()


def kernel(x, Ws, rs):
    x32 = x.astype(jnp.int32)
    xf = lax.bitcast_convert_type(x32, jnp.float32)
    Wt = Ws.transpose(1, 2, 0).reshape(D, N_DISTS)
    rt = rs.transpose(1, 0)
    OW, OR = _gather(x32, xf, Wt, rt)
    W = OW.reshape(HIDDEN, N_BITS, B).transpose(2, 0, 1)
    r = OR.transpose(1, 0)
    return (W, r)
